# baseline (speedup 1.0000x reference)
import functools

import jax
import jax.numpy as jnp
import numpy as np
from jax import lax
from jax.experimental import pallas as pl
from jax.experimental.pallas import tpu as pltpu
from jax.experimental.pallas import tpu_sc as plsc

N = 10000
E = 320000
H = 96
NC = 2
NS = 16
NW = NC * NS
EW = E // NW
CH = 80
NCHUNK = EW // CH
NROW = N // NS

TE = 2560
GE = E // TE
TN = 2000
GN = N // TN


_F = np.float32


def _erfc(z):
    az = jnp.abs(z)
    z2 = z * z
    p = z2 * _F(7.85386146e-05)
    p = p + _F(-0.000801019371)
    p = p * z2
    p = p + _F(0.00518832775)
    p = p * z2
    p = p + _F(-0.0268538129)
    p = p * z2
    p = p + _F(0.112835854)
    p = p * z2
    p = p + _F(-0.37612626)
    p = p * z2
    p = p + _F(1.12837911)
    small = _F(1.0) - z * p

    nz2 = -z2
    e = jnp.exp(nz2)
    r = e * (_F(1.0) / az)
    w = _F(1.0) / z2
    p1 = w * _F(0.0232682)
    p1 = p1 + _F(-0.138703942)
    p1 = p1 * w
    p1 = p1 + _F(0.368742466)
    p1 = p1 * w
    p1 = p1 + _F(-0.582473278)
    p1 = p1 * w
    p1 = p1 + _F(0.621000469)
    p1 = p1 * w
    p1 = p1 + _F(-0.494451523)
    p1 = p1 * w
    p1 = p1 + _F(0.340488)
    p1 = p1 * w
    p1 = p1 + _F(-0.274112701)
    p1 = p1 * w
    p1 = p1 + _F(0.563825965)
    p2 = w * _F(-10.477664)
    p2 = p2 + _F(12.9772)
    p2 = p2 * w
    p2 = p2 + _F(-7.49551868)
    p2 = p2 * w
    p2 = p2 + _F(2.92101908)
    p2 = p2 * w
    p2 = p2 + _F(-1.01526523)
    p2 = p2 * w
    p2 = p2 + _F(0.42184633)
    p2 = p2 * w
    p2 = p2 + _F(-0.282076746)
    p2 = p2 * w
    p2 = p2 + _F(0.564189494)
    sel = jnp.where(az < _F(2.0), p1, p2)
    res = r * sel
    res = jnp.where(nz2 < _F(-88.7228394), _F(0.0), res)
    res = jnp.where(z < _F(0.0), _F(2.0) - res, res)
    return jnp.where(az < _F(1.0), small, res)


def _gelu(x):
    return (x * _F(0.5)) * _erfc(-x * _F(0.707106769))


def _lane_sum(x):
    acc = x[:, 0:8]
    for k in range(1, 12):
        acc = acc + x[:, 8 * k:8 * k + 8]
    u = acc[:, :4] + acc[:, 4:]
    v = u[:, :2] + u[:, 2:]
    return v[:, :1] + v[:, 1:]


def _ln(y, gam, bet):
    inv_n = _F(0.010416667)
    mu = _lane_sum(y) * inv_n
    d = y - mu
    var = _lane_sum(d * d) * inv_n
    return d / jnp.sqrt(var + _F(1e-5)) * gam + bet


def _dot(a, b):
    return jnp.dot(a.astype(jnp.bfloat16), b.astype(jnp.bfloat16),
                   preferred_element_type=jnp.float32)


def _dot288(a, b):
    return _dot(a[:, :144], b[:144]) + _dot(a[:, 144:], b[144:])


_sc_mesh = plsc.VectorSubcoreMesh(
    core_axis_name="c", subcore_axis_name="s", num_cores=NC, num_subcores=NS)

_sc_params = pltpu.CompilerParams(use_tc_tiling_on_sc=False,
                                  needs_layout_passes=False)


@functools.partial(
    pl.kernel,
    out_type=(jax.ShapeDtypeStruct((E, H), jnp.float32),
              jax.ShapeDtypeStruct((E, H), jnp.float32)),
    mesh=_sc_mesh,
    scratch_types=[
        pltpu.VMEM((NCHUNK, CH), jnp.int32),
        pltpu.VMEM((NCHUNK, CH), jnp.int32),
        pltpu.VMEM((CH, H), jnp.float32),
        pltpu.VMEM((CH, H), jnp.float32),
        pltpu.VMEM((CH, H), jnp.float32),
        pltpu.VMEM((CH, H), jnp.float32),
        pltpu.VMEM((CH, H), jnp.float32),
        pltpu.VMEM((CH, H), jnp.float32),
        pltpu.VMEM((CH, H), jnp.float32),
        pltpu.VMEM((CH, H), jnp.float32),
        pltpu.SemaphoreType.DMA,
        pltpu.SemaphoreType.DMA,
        pltpu.SemaphoreType.DMA,
        pltpu.SemaphoreType.DMA,
        pltpu.SemaphoreType.DMA,
        pltpu.SemaphoreType.DMA,
        pltpu.SemaphoreType.DMA,
        pltpu.SemaphoreType.DMA,
        pltpu.SemaphoreType.DMA,
    ],
    compiler_params=_sc_params,
)
def _gather_two(h_hbm, src3_hbm, dst3_hbm, hs_hbm, hd_hbm,
                si_v, di_v, ra0_v, rb0_v, ra1_v, rb1_v, ra2_v, rb2_v,
                ra3_v, rb3_v,
                sa0, sb0, sa1, sb1, sa2, sb2, sa3, sb3, sem_i):
    wid = lax.axis_index("s") * NC + lax.axis_index("c")
    base = wid * EW

    pltpu.async_copy(src3_hbm.at[wid], si_v, sem_i).wait()
    pltpu.async_copy(dst3_hbm.at[wid], di_v, sem_i).wait()

    slots = ((ra0_v, rb0_v, sa0, sb0),
             (ra1_v, rb1_v, sa1, sb1),
             (ra2_v, rb2_v, sa2, sb2),
             (ra3_v, rb3_v, sa3, sb3))

    def fetch(k, slot):
        ra, rb, sa, sb = slot
        pltpu.async_copy(h_hbm.at[si_v.at[k]], ra, sa)
        pltpu.async_copy(h_hbm.at[di_v.at[k]], rb, sb)

    def drain_store(k, slot):
        ra, rb, sa, sb = slot
        dst = pl.ds(base + k * CH, CH)
        pltpu.make_async_copy(ra, hs_hbm.at[dst], sa).wait()
        pltpu.make_async_copy(rb, hd_hbm.at[dst], sb).wait()

    def step(k, bcur, bpre, drain, refetch):
        ra, rb, sa, sb = slots[bcur]
        pltpu.make_async_copy(h_hbm.at[si_v.at[k]], ra, sa).wait()
        pltpu.make_async_copy(h_hbm.at[di_v.at[k]], rb, sb).wait()
        dst = pl.ds(base + k * CH, CH)
        pltpu.async_copy(ra, hs_hbm.at[dst], sa)
        pltpu.async_copy(rb, hd_hbm.at[dst], sb)
        if drain:
            drain_store(k - 2, slots[bpre])
        if refetch:
            fetch(k + 2, slots[bpre])

    fetch(0, slots[0])
    fetch(1, slots[1])
    step(0, 0, 2, False, True)
    step(1, 1, 3, False, True)

    def quad(j, carry):
        for b in range(4):
            step(4 * j + 2 + b, (2 + b) % 4, b, True, True)
        return carry

    lax.fori_loop(0, (NCHUNK - 5) // 4, quad, 0)
    step(NCHUNK - 3, 2, 0, True, True)
    step(NCHUNK - 2, 3, 1, True, False)
    step(NCHUNK - 1, 0, 2, True, False)
    drain_store(NCHUNK - 2, slots[3])
    drain_store(NCHUNK - 1, slots[0])


TSC = 2560
NT = E // TSC
NRW = 313
TRASH = TSC + CH


@functools.partial(
    pl.kernel,
    out_type=jax.ShapeDtypeStruct((N, H), jnp.float32),
    mesh=_sc_mesh,
    scratch_types=[
        pltpu.VMEM((TSC,), jnp.int32),
        pltpu.VMEM((TSC + CH + 16,), jnp.int32),
        pltpu.VMEM((TSC + CH + 16,), jnp.int32),
        pltpu.VMEM((CH,), jnp.int32),
        pltpu.VMEM((CH,), jnp.int32),
        pltpu.VMEM((CH, H), jnp.float32),
        pltpu.VMEM((CH, H), jnp.float32),
        pltpu.VMEM_SHARED((NS * (NRW + 1), H), jnp.float32),
        pltpu.SemaphoreType.DMA,
        pltpu.SemaphoreType.DMA,
    ],
    compiler_params=_sc_params,
)
def _scatter_sum(e_hbm, dst2_hbm, out_hbm,
                 dt_v, se_v, sr_v, eid_v, r_v, er_v, z_v, acc_sh,
                 sem_d, sem_e):
    cid = lax.axis_index("c")
    sid = lax.axis_index("s")
    wid = sid * NC + cid
    lo = wid * 312 + jnp.minimum(wid, 16)
    arow = sid * (NRW + 1)

    zero16 = jnp.zeros((16,), jnp.float32)
    zi = jnp.zeros((16,), jnp.int32)
    iota16 = lax.iota(jnp.int32, 16)
    lo_v = zi + lo
    hi_v = lo_v + jnp.where(wid < 16, 313, 312)
    arow_v = zi + arow
    dump_v = arow_v + NRW

    def zrow(r, carry):
        for j in range(H // 16):
            z_v[r, pl.ds(j * 16, 16)] = zero16
        return carry

    lax.fori_loop(0, CH, zrow, 0)
    for q in range(3):
        pltpu.sync_copy(z_v, acc_sh.at[pl.ds(arow + q * CH, CH)])
    pltpu.sync_copy(z_v.at[pl.ds(0, NRW + 1 - 3 * CH)],
                    acc_sh.at[pl.ds(arow + 3 * CH, NRW + 1 - 3 * CH)])

    def tile(t, carry):
        pltpu.async_copy(dst2_hbm.at[t], dt_v, sem_d).wait()

        def vec(v, off):
            d = dt_v[pl.ds(v * 16, 16)]
            m = (d >= lo_v) & (d < hi_v)
            mi = m.astype(jnp.int32)
            eid = (zi + (t * TSC + v * 16)) + iota16
            c = plsc.cumsum(mi)
            p = jnp.where(m, (zi + off) + c - 1, (zi + TRASH) + iota16)
            plsc.store_scatter(se_v, [p], eid)
            plsc.store_scatter(sr_v, [p], (d - lo_v) + arow_v)
            return off + jnp.sum(mi, axis=0)

        off = lax.fori_loop(0, TSC // 16, vec, 0)
        for j in range(CH // 16):
            pad = (zi + (off + j * 16)) + iota16
            plsc.store_scatter(se_v, [pad], zi)
            plsc.store_scatter(sr_v, [pad], dump_v)

        nch = (off + CH - 1) // CH

        def chunk(c, carry2):
            for j in range(CH // 16):
                sl = pl.ds(j * 16, 16)
                eid_v[sl] = se_v[pl.ds(c * CH + j * 16, 16)]
                r_v[sl] = sr_v[pl.ds(c * CH + j * 16, 16)]
            pltpu.async_copy(e_hbm.at[eid_v], er_v, sem_e).wait()
            pltpu.sync_copy(er_v, acc_sh.at[r_v], add=True)
            return carry2

        lax.fori_loop(0, nch, chunk, 0)
        return carry

    lax.fori_loop(0, NT, tile, 0)

    @pl.when(wid < 16)
    def _():
        pltpu.sync_copy(acc_sh.at[pl.ds(arow, 313)],
                        out_hbm.at[pl.ds(lo, 313)])

    @pl.when(wid >= 16)
    def _():
        pltpu.sync_copy(acc_sh.at[pl.ds(arow, 312)],
                        out_hbm.at[pl.ds(lo, 312)])


def _vec_spec():
    return pl.BlockSpec((1, H), lambda i: (0, 0))


def _mat_spec(d0=H, d1=H):
    return pl.BlockSpec((d0, d1), lambda i: (0, 0))


def _edge_enc_body(ea_ref, w1_ref, b1_ref, w2_ref, b2_ref, out_ref):
    t = _dot(ea_ref[...], w1_ref[...]) + b1_ref[...]
    out_ref[...] = _dot(_gelu(t), w2_ref[...]) + b2_ref[...]


_edge_enc = pl.pallas_call(
    _edge_enc_body,
    grid=(GE,),
    in_specs=[
        pl.BlockSpec((TE, 8), lambda i: (i, 0)),
        pl.BlockSpec((8, H), lambda i: (0, 0)),
        _vec_spec(),
        _mat_spec(),
        _vec_spec(),
    ],
    out_specs=pl.BlockSpec((TE, H), lambda i: (i, 0)),
    out_shape=jax.ShapeDtypeStruct((E, H), jnp.float32),
)


def _node_enc_body(x_ref, w1_ref, b1_ref, w2_ref, b2_ref, h_ref):
    t = _dot(x_ref[...], w1_ref[...]) + b1_ref[...]
    h_ref[...] = _dot(_gelu(t), w2_ref[...]) + b2_ref[...]


_node_enc = pl.pallas_call(
    _node_enc_body,
    grid=(GN,),
    in_specs=[
        pl.BlockSpec((TN, 8), lambda i: (i, 0)),
        pl.BlockSpec((8, H), lambda i: (0, 0)),
        _vec_spec(),
        _mat_spec(),
        _vec_spec(),
    ],
    out_specs=pl.BlockSpec((TN, H), lambda i: (i, 0)),
    out_shape=jax.ShapeDtypeStruct((N, H), jnp.float32),
)


def _edge_update_body(e_ref, hs_ref, hd_ref, w1_ref, b1_ref, w2_ref, b2_ref,
                      gam_ref, bet_ref, out_ref):
    e = e_ref[...]
    e_in = jnp.concatenate([hs_ref[...], hd_ref[...], e], axis=-1)
    t = _dot288(e_in, w1_ref[...]) + b1_ref[...]
    y = e + (_dot(_gelu(t), w2_ref[...]) + b2_ref[...])
    out_ref[...] = _ln(y, gam_ref[...], bet_ref[...])


_edge_update = pl.pallas_call(
    _edge_update_body,
    grid=(GE,),
    in_specs=[
        pl.BlockSpec((TE, H), lambda i: (i, 0)),
        pl.BlockSpec((TE, H), lambda i: (i, 0)),
        pl.BlockSpec((TE, H), lambda i: (i, 0)),
        _mat_spec(3 * H, H),
        _vec_spec(),
        _mat_spec(),
        _vec_spec(),
        _vec_spec(),
        _vec_spec(),
    ],
    out_specs=pl.BlockSpec((TE, H), lambda i: (i, 0)),
    out_shape=jax.ShapeDtypeStruct((E, H), jnp.float32),
)


def _node_update_body(h_ref, g0_ref, v1_ref, vb1_ref, v2_ref,
                      vb2_ref, gam_ref, bet_ref, h_out):
    h = h_ref[...]
    agg = g0_ref[...]
    t = _dot(jnp.concatenate([h, agg], axis=-1), v1_ref[...]) + vb1_ref[...]
    y = h + (_dot(_gelu(t), v2_ref[...]) + vb2_ref[...])
    h_out[...] = _ln(y, gam_ref[...], bet_ref[...])


_node_update = pl.pallas_call(
    _node_update_body,
    grid=(GN,),
    in_specs=[
        pl.BlockSpec((TN, H), lambda i: (i, 0)),
        pl.BlockSpec((TN, H), lambda i: (i, 0)),
        _mat_spec(2 * H, H),
        _vec_spec(),
        _mat_spec(),
        _vec_spec(),
        _vec_spec(),
        _vec_spec(),
    ],
    out_specs=pl.BlockSpec((TN, H), lambda i: (i, 0)),
    out_shape=jax.ShapeDtypeStruct((N, H), jnp.float32),
)


def _node_last_body(h_ref, g0_ref, v1_ref, vb1_ref, v2_ref,
                    vb2_ref, gam_ref, bet_ref,
                    nw1_ref, nb1_ref, nw2_ref, nb2_ref,
                    h_out, nl_out):
    h = h_ref[...]
    agg = g0_ref[...]
    t = _dot(jnp.concatenate([h, agg], axis=-1), v1_ref[...]) + vb1_ref[...]
    y = h + (_dot(_gelu(t), v2_ref[...]) + vb2_ref[...])
    hn = _ln(y, gam_ref[...], bet_ref[...])
    h_out[...] = hn
    u = _gelu(_dot(hn, nw1_ref[...]) + nb1_ref[...])
    nl_out[...] = _dot(u, nw2_ref[...]) + nb2_ref[...]


_node_last = pl.pallas_call(
    _node_last_body,
    grid=(GN,),
    in_specs=[
        pl.BlockSpec((TN, H), lambda i: (i, 0)),
        pl.BlockSpec((TN, H), lambda i: (i, 0)),
        _mat_spec(2 * H, H),
        _vec_spec(),
        _mat_spec(),
        _vec_spec(),
        _vec_spec(),
        _vec_spec(),
        _mat_spec(),
        _vec_spec(),
        pl.BlockSpec((H, 1), lambda i: (0, 0)),
        pl.BlockSpec((1, 1), lambda i: (0, 0)),
    ],
    out_specs=[
        pl.BlockSpec((TN, H), lambda i: (i, 0)),
        pl.BlockSpec((TN, 1), lambda i: (i, 0)),
    ],
    out_shape=[
        jax.ShapeDtypeStruct((N, H), jnp.float32),
        jax.ShapeDtypeStruct((N, 1), jnp.float32),
    ],
)


def _edge_head_body(e_ref, hs_ref, hd_ref, w1_ref, b1_ref, w2_ref, b2_ref,
                    out_ref):
    e_in = jnp.concatenate([hs_ref[...], hd_ref[...], e_ref[...]], axis=-1)
    t = _dot288(e_in, w1_ref[...]) + b1_ref[...]
    out_ref[...] = _dot(_gelu(t), w2_ref[...]) + b2_ref[...]


_edge_head = pl.pallas_call(
    _edge_head_body,
    grid=(GE,),
    in_specs=[
        pl.BlockSpec((TE, H), lambda i: (i, 0)),
        pl.BlockSpec((TE, H), lambda i: (i, 0)),
        pl.BlockSpec((TE, H), lambda i: (i, 0)),
        _mat_spec(3 * H, H),
        _vec_spec(),
        pl.BlockSpec((H, 1), lambda i: (0, 0)),
        pl.BlockSpec((1, 1), lambda i: (0, 0)),
    ],
    out_specs=pl.BlockSpec((TE, 1), lambda i: (i, 0)),
    out_shape=jax.ShapeDtypeStruct((E, 1), jnp.float32),
)


def _row(v):
    return v.reshape(1, -1)


def kernel(x, edge_index, edge_attr, params):
    src = edge_index[0]
    dst = edge_index[1]
    src3 = src.reshape(NW, NCHUNK, CH)
    dst3 = dst.reshape(NW, NCHUNK, CH)
    dst2 = dst.reshape(NT, TSC)

    xp = jnp.pad(x, ((0, 0), (0, 2)))
    ne = params["node_enc"]
    nw1 = jnp.pad(ne["l1"]["w"], ((0, 2), (0, 0)))
    h = _node_enc(xp, nw1, _row(ne["l1"]["b"]),
                  ne["l2"]["w"], _row(ne["l2"]["b"]))

    ee = params["edge_enc"]
    e = _edge_enc(edge_attr, ee["l1"]["w"], _row(ee["l1"]["b"]),
                  ee["l2"]["w"], _row(ee["l2"]["b"]))

    blocks = params["blocks"]
    for k, bp in enumerate(blocks):
        hs, hd = _gather_two(h, src3, dst3)
        em = bp["edge_mlp"]
        e = _edge_update(e, hs, hd,
                         em["l1"]["w"], _row(em["l1"]["b"]),
                         em["l2"]["w"], _row(em["l2"]["b"]),
                         _row(bp["ln_e"]["g"]), _row(bp["ln_e"]["b"]))
        agg = _scatter_sum(e, dst2)
        nm = bp["node_mlp"]
        common = (h, agg,
                  nm["l1"]["w"], _row(nm["l1"]["b"]),
                  nm["l2"]["w"], _row(nm["l2"]["b"]),
                  _row(bp["ln_h"]["g"]), _row(bp["ln_h"]["b"]))
        if k + 1 < len(blocks):
            h = _node_update(*common)
        else:
            nh = params["node_head"]
            h, nl = _node_last(
                *common,
                nh["l1"]["w"], _row(nh["l1"]["b"]),
                nh["l2"]["w"], _row(nh["l2"]["b"]))

    hs, hd = _gather_two(h, src3, dst3)
    hd_p = params["edge_head"]
    el = _edge_head(e, hs, hd,
                    hd_p["l1"]["w"], _row(hd_p["l1"]["b"]),
                    hd_p["l2"]["w"], _row(hd_p["l2"]["b"]))

    return (el[:, 0], nl[:, 0])


# speedup vs baseline: 2.4099x; 2.4099x over previous
import functools

import jax
import jax.numpy as jnp
import numpy as np
from jax import lax
from jax.experimental import pallas as pl
from jax.experimental.pallas import tpu as pltpu
from jax.experimental.pallas import tpu_sc as plsc

N = 10000
E = 320000
H = 96
NC = 2
NS = 16
NW = NC * NS
EW = E // NW
CH = 80
NCHUNK = EW // CH
NROW = N // NS

TE = 2560
GE = E // TE
TN = 2000
GN = N // TN


_F = np.float32


def _erfc(z):
    az = jnp.abs(z)
    z2 = z * z
    p = z2 * _F(7.85386146e-05)
    p = p + _F(-0.000801019371)
    p = p * z2
    p = p + _F(0.00518832775)
    p = p * z2
    p = p + _F(-0.0268538129)
    p = p * z2
    p = p + _F(0.112835854)
    p = p * z2
    p = p + _F(-0.37612626)
    p = p * z2
    p = p + _F(1.12837911)
    small = _F(1.0) - z * p

    nz2 = -z2
    e = jnp.exp(nz2)
    r = e * (_F(1.0) / az)
    w = _F(1.0) / z2
    p1 = w * _F(0.0232682)
    p1 = p1 + _F(-0.138703942)
    p1 = p1 * w
    p1 = p1 + _F(0.368742466)
    p1 = p1 * w
    p1 = p1 + _F(-0.582473278)
    p1 = p1 * w
    p1 = p1 + _F(0.621000469)
    p1 = p1 * w
    p1 = p1 + _F(-0.494451523)
    p1 = p1 * w
    p1 = p1 + _F(0.340488)
    p1 = p1 * w
    p1 = p1 + _F(-0.274112701)
    p1 = p1 * w
    p1 = p1 + _F(0.563825965)
    p2 = w * _F(-10.477664)
    p2 = p2 + _F(12.9772)
    p2 = p2 * w
    p2 = p2 + _F(-7.49551868)
    p2 = p2 * w
    p2 = p2 + _F(2.92101908)
    p2 = p2 * w
    p2 = p2 + _F(-1.01526523)
    p2 = p2 * w
    p2 = p2 + _F(0.42184633)
    p2 = p2 * w
    p2 = p2 + _F(-0.282076746)
    p2 = p2 * w
    p2 = p2 + _F(0.564189494)
    sel = jnp.where(az < _F(2.0), p1, p2)
    res = r * sel
    res = jnp.where(nz2 < _F(-88.7228394), _F(0.0), res)
    res = jnp.where(z < _F(0.0), _F(2.0) - res, res)
    return jnp.where(az < _F(1.0), small, res)


def _gelu(x):
    return (x * _F(0.5)) * _erfc(-x * _F(0.707106769))


def _lane_sum(x):
    acc = x[:, 0:8]
    for k in range(1, 12):
        acc = acc + x[:, 8 * k:8 * k + 8]
    u = acc[:, :4] + acc[:, 4:]
    v = u[:, :2] + u[:, 2:]
    return v[:, :1] + v[:, 1:]


def _ln(y, gam, bet):
    inv_n = _F(0.010416667)
    mu = _lane_sum(y) * inv_n
    d = y - mu
    var = _lane_sum(d * d) * inv_n
    return d / jnp.sqrt(var + _F(1e-5)) * gam + bet


def _dot(a, b):
    return jnp.dot(a.astype(jnp.bfloat16), b.astype(jnp.bfloat16),
                   preferred_element_type=jnp.float32)


def _dot288(a, b):
    return _dot(a[:, :144], b[:144]) + _dot(a[:, 144:], b[144:])


_sc_mesh = plsc.VectorSubcoreMesh(
    core_axis_name="c", subcore_axis_name="s", num_cores=NC, num_subcores=NS)

_sc_params = pltpu.CompilerParams(use_tc_tiling_on_sc=False,
                                  needs_layout_passes=False)


@functools.partial(
    pl.kernel,
    out_type=(jax.ShapeDtypeStruct((E, H), jnp.float32),
              jax.ShapeDtypeStruct((E, H), jnp.float32)),
    mesh=_sc_mesh,
    scratch_types=[
        pltpu.VMEM((NCHUNK, CH), jnp.int32),
        pltpu.VMEM((NCHUNK, CH), jnp.int32),
        pltpu.VMEM((CH, H), jnp.float32),
        pltpu.VMEM((CH, H), jnp.float32),
        pltpu.VMEM((CH, H), jnp.float32),
        pltpu.VMEM((CH, H), jnp.float32),
        pltpu.VMEM((CH, H), jnp.float32),
        pltpu.VMEM((CH, H), jnp.float32),
        pltpu.VMEM((CH, H), jnp.float32),
        pltpu.VMEM((CH, H), jnp.float32),
        pltpu.SemaphoreType.DMA,
        pltpu.SemaphoreType.DMA,
        pltpu.SemaphoreType.DMA,
        pltpu.SemaphoreType.DMA,
        pltpu.SemaphoreType.DMA,
        pltpu.SemaphoreType.DMA,
        pltpu.SemaphoreType.DMA,
        pltpu.SemaphoreType.DMA,
        pltpu.SemaphoreType.DMA,
    ],
    compiler_params=_sc_params,
)
def _gather_two(h_hbm, src3_hbm, dst3_hbm, hs_hbm, hd_hbm,
                si_v, di_v, ra0_v, rb0_v, ra1_v, rb1_v, ra2_v, rb2_v,
                ra3_v, rb3_v,
                sa0, sb0, sa1, sb1, sa2, sb2, sa3, sb3, sem_i):
    wid = lax.axis_index("s") * NC + lax.axis_index("c")
    base = wid * EW

    pltpu.async_copy(src3_hbm.at[wid], si_v, sem_i).wait()
    pltpu.async_copy(dst3_hbm.at[wid], di_v, sem_i).wait()

    slots = ((ra0_v, rb0_v, sa0, sb0),
             (ra1_v, rb1_v, sa1, sb1),
             (ra2_v, rb2_v, sa2, sb2),
             (ra3_v, rb3_v, sa3, sb3))

    def fetch(k, slot):
        ra, rb, sa, sb = slot
        pltpu.async_copy(h_hbm.at[si_v.at[k]], ra, sa)
        pltpu.async_copy(h_hbm.at[di_v.at[k]], rb, sb)

    def drain_store(k, slot):
        ra, rb, sa, sb = slot
        dst = pl.ds(base + k * CH, CH)
        pltpu.make_async_copy(ra, hs_hbm.at[dst], sa).wait()
        pltpu.make_async_copy(rb, hd_hbm.at[dst], sb).wait()

    def step(k, bcur, bpre, drain, refetch):
        ra, rb, sa, sb = slots[bcur]
        pltpu.make_async_copy(h_hbm.at[si_v.at[k]], ra, sa).wait()
        pltpu.make_async_copy(h_hbm.at[di_v.at[k]], rb, sb).wait()
        dst = pl.ds(base + k * CH, CH)
        pltpu.async_copy(ra, hs_hbm.at[dst], sa)
        pltpu.async_copy(rb, hd_hbm.at[dst], sb)
        if drain:
            drain_store(k - 2, slots[bpre])
        if refetch:
            fetch(k + 2, slots[bpre])

    fetch(0, slots[0])
    fetch(1, slots[1])
    step(0, 0, 2, False, True)
    step(1, 1, 3, False, True)

    def quad(j, carry):
        for b in range(4):
            step(4 * j + 2 + b, (2 + b) % 4, b, True, True)
        return carry

    lax.fori_loop(0, (NCHUNK - 5) // 4, quad, 0)
    step(NCHUNK - 3, 2, 0, True, True)
    step(NCHUNK - 2, 3, 1, True, False)
    step(NCHUNK - 1, 0, 2, True, False)
    drain_store(NCHUNK - 2, slots[3])
    drain_store(NCHUNK - 1, slots[0])


@functools.partial(
    pl.kernel,
    out_type=jax.ShapeDtypeStruct((NC, N, H), jnp.float32),
    mesh=_sc_mesh,
    scratch_types=[
        pltpu.VMEM((NCHUNK, CH), jnp.int32),
        pltpu.VMEM((CH, H), jnp.float32),
        pltpu.VMEM((CH, H), jnp.float32),
        pltpu.VMEM((NROW // 5, H), jnp.float32),
        pltpu.VMEM_SHARED((N, H), jnp.float32),
        pltpu.SemaphoreType.DMA,
        pltpu.SemaphoreType.DMA,
        pltpu.SemaphoreType.DMA,
    ],
    compiler_params=_sc_params,
)
def _scatter_sum(e_hbm, dst3_hbm, out_hbm, di_v, er0_v, er1_v, z_v, acc_sh,
                 se0, se1, sem_i):
    cid = lax.axis_index("c")
    sid = lax.axis_index("s")
    wid = sid * NC + cid
    base = wid * EW

    cpi = pltpu.async_copy(dst3_hbm.at[wid], di_v, sem_i)

    zero = jnp.zeros((16,), jnp.float32)

    def zrow(r, carry):
        for j in range(H // 16):
            z_v[r, pl.ds(j * 16, 16)] = zero
        return carry

    lax.fori_loop(0, NROW // 5, zrow, 0)
    for q in range(5):
        pltpu.sync_copy(
            z_v, acc_sh.at[pl.ds(sid * NROW + q * (NROW // 5), NROW // 5)])
    cpi.wait()
    plsc.subcore_barrier()

    slots = ((er0_v, se0), (er1_v, se1))

    def fetch(k, slot):
        er, se = slot
        pltpu.async_copy(e_hbm.at[pl.ds(base + k * CH, CH)], er, se)

    def process(k, slot):
        er, se = slot
        pltpu.make_async_copy(e_hbm.at[pl.ds(base + k * CH, CH)], er,
                              se).wait()
        pltpu.sync_copy(er, acc_sh.at[di_v.at[k]], add=True)

    fetch(0, slots[0])

    def pair(j, carry):
        for b in range(2):
            k = 2 * j + b
            fetch(k + 1, slots[1 - b])
            process(k, slots[b])
        return carry

    lax.fori_loop(0, (NCHUNK - 1) // 2, pair, 0)
    process(NCHUNK - 1, slots[0])
    plsc.subcore_barrier()
    pltpu.sync_copy(acc_sh.at[pl.ds(sid * NROW, NROW)],
                    out_hbm.at[cid, pl.ds(sid * NROW, NROW)])


def _vec_spec():
    return pl.BlockSpec((1, H), lambda i: (0, 0))


def _mat_spec(d0=H, d1=H):
    return pl.BlockSpec((d0, d1), lambda i: (0, 0))


def _edge_enc_body(ea_ref, w1_ref, b1_ref, w2_ref, b2_ref, out_ref):
    t = _dot(ea_ref[...], w1_ref[...]) + b1_ref[...]
    out_ref[...] = _dot(_gelu(t), w2_ref[...]) + b2_ref[...]


_edge_enc = pl.pallas_call(
    _edge_enc_body,
    grid=(GE,),
    in_specs=[
        pl.BlockSpec((TE, 8), lambda i: (i, 0)),
        pl.BlockSpec((8, H), lambda i: (0, 0)),
        _vec_spec(),
        _mat_spec(),
        _vec_spec(),
    ],
    out_specs=pl.BlockSpec((TE, H), lambda i: (i, 0)),
    out_shape=jax.ShapeDtypeStruct((E, H), jnp.float32),
)


def _node_enc_body(x_ref, w1_ref, b1_ref, w2_ref, b2_ref, h_ref):
    t = _dot(x_ref[...], w1_ref[...]) + b1_ref[...]
    h_ref[...] = _dot(_gelu(t), w2_ref[...]) + b2_ref[...]


_node_enc = pl.pallas_call(
    _node_enc_body,
    grid=(GN,),
    in_specs=[
        pl.BlockSpec((TN, 8), lambda i: (i, 0)),
        pl.BlockSpec((8, H), lambda i: (0, 0)),
        _vec_spec(),
        _mat_spec(),
        _vec_spec(),
    ],
    out_specs=pl.BlockSpec((TN, H), lambda i: (i, 0)),
    out_shape=jax.ShapeDtypeStruct((N, H), jnp.float32),
)


def _edge_update_body(e_ref, hs_ref, hd_ref, w1_ref, b1_ref, w2_ref, b2_ref,
                      gam_ref, bet_ref, out_ref):
    e = e_ref[...]
    e_in = jnp.concatenate([hs_ref[...], hd_ref[...], e], axis=-1)
    t = _dot288(e_in, w1_ref[...]) + b1_ref[...]
    y = e + (_dot(_gelu(t), w2_ref[...]) + b2_ref[...])
    out_ref[...] = _ln(y, gam_ref[...], bet_ref[...])


_edge_update = pl.pallas_call(
    _edge_update_body,
    grid=(GE,),
    in_specs=[
        pl.BlockSpec((TE, H), lambda i: (i, 0)),
        pl.BlockSpec((TE, H), lambda i: (i, 0)),
        pl.BlockSpec((TE, H), lambda i: (i, 0)),
        _mat_spec(3 * H, H),
        _vec_spec(),
        _mat_spec(),
        _vec_spec(),
        _vec_spec(),
        _vec_spec(),
    ],
    out_specs=pl.BlockSpec((TE, H), lambda i: (i, 0)),
    out_shape=jax.ShapeDtypeStruct((E, H), jnp.float32),
)


def _node_update_body(h_ref, g0_ref, g1_ref, v1_ref, vb1_ref, v2_ref,
                      vb2_ref, gam_ref, bet_ref, h_out):
    h = h_ref[...]
    agg = g0_ref[...] + g1_ref[...]
    t = _dot(jnp.concatenate([h, agg], axis=-1), v1_ref[...]) + vb1_ref[...]
    y = h + (_dot(_gelu(t), v2_ref[...]) + vb2_ref[...])
    h_out[...] = _ln(y, gam_ref[...], bet_ref[...])


_node_update = pl.pallas_call(
    _node_update_body,
    grid=(GN,),
    in_specs=[
        pl.BlockSpec((TN, H), lambda i: (i, 0)),
        pl.BlockSpec((TN, H), lambda i: (i, 0)),
        pl.BlockSpec((TN, H), lambda i: (i, 0)),
        _mat_spec(2 * H, H),
        _vec_spec(),
        _mat_spec(),
        _vec_spec(),
        _vec_spec(),
        _vec_spec(),
    ],
    out_specs=pl.BlockSpec((TN, H), lambda i: (i, 0)),
    out_shape=jax.ShapeDtypeStruct((N, H), jnp.float32),
)


def _node_last_body(h_ref, g0_ref, g1_ref, v1_ref, vb1_ref, v2_ref,
                    vb2_ref, gam_ref, bet_ref,
                    nw1_ref, nb1_ref, nw2_ref, nb2_ref,
                    h_out, nl_out):
    h = h_ref[...]
    agg = g0_ref[...] + g1_ref[...]
    t = _dot(jnp.concatenate([h, agg], axis=-1), v1_ref[...]) + vb1_ref[...]
    y = h + (_dot(_gelu(t), v2_ref[...]) + vb2_ref[...])
    hn = _ln(y, gam_ref[...], bet_ref[...])
    h_out[...] = hn
    u = _gelu(_dot(hn, nw1_ref[...]) + nb1_ref[...])
    nl_out[...] = _dot(u, nw2_ref[...]) + nb2_ref[...]


_node_last = pl.pallas_call(
    _node_last_body,
    grid=(GN,),
    in_specs=[
        pl.BlockSpec((TN, H), lambda i: (i, 0)),
        pl.BlockSpec((TN, H), lambda i: (i, 0)),
        pl.BlockSpec((TN, H), lambda i: (i, 0)),
        _mat_spec(2 * H, H),
        _vec_spec(),
        _mat_spec(),
        _vec_spec(),
        _vec_spec(),
        _vec_spec(),
        _mat_spec(),
        _vec_spec(),
        pl.BlockSpec((H, 1), lambda i: (0, 0)),
        pl.BlockSpec((1, 1), lambda i: (0, 0)),
    ],
    out_specs=[
        pl.BlockSpec((TN, H), lambda i: (i, 0)),
        pl.BlockSpec((TN, 1), lambda i: (i, 0)),
    ],
    out_shape=[
        jax.ShapeDtypeStruct((N, H), jnp.float32),
        jax.ShapeDtypeStruct((N, 1), jnp.float32),
    ],
)


def _edge_head_body(e_ref, hs_ref, hd_ref, w1_ref, b1_ref, w2_ref, b2_ref,
                    out_ref):
    e_in = jnp.concatenate([hs_ref[...], hd_ref[...], e_ref[...]], axis=-1)
    t = _dot288(e_in, w1_ref[...]) + b1_ref[...]
    out_ref[...] = _dot(_gelu(t), w2_ref[...]) + b2_ref[...]


_edge_head = pl.pallas_call(
    _edge_head_body,
    grid=(GE,),
    in_specs=[
        pl.BlockSpec((TE, H), lambda i: (i, 0)),
        pl.BlockSpec((TE, H), lambda i: (i, 0)),
        pl.BlockSpec((TE, H), lambda i: (i, 0)),
        _mat_spec(3 * H, H),
        _vec_spec(),
        pl.BlockSpec((H, 1), lambda i: (0, 0)),
        pl.BlockSpec((1, 1), lambda i: (0, 0)),
    ],
    out_specs=pl.BlockSpec((TE, 1), lambda i: (i, 0)),
    out_shape=jax.ShapeDtypeStruct((E, 1), jnp.float32),
)


def _row(v):
    return v.reshape(1, -1)


def kernel(x, edge_index, edge_attr, params):
    src = edge_index[0]
    dst = edge_index[1]
    src3 = src.reshape(NW, NCHUNK, CH)
    dst3 = dst.reshape(NW, NCHUNK, CH)

    xp = jnp.pad(x, ((0, 0), (0, 2)))
    ne = params["node_enc"]
    nw1 = jnp.pad(ne["l1"]["w"], ((0, 2), (0, 0)))
    h = _node_enc(xp, nw1, _row(ne["l1"]["b"]),
                  ne["l2"]["w"], _row(ne["l2"]["b"]))

    ee = params["edge_enc"]
    e = _edge_enc(edge_attr, ee["l1"]["w"], _row(ee["l1"]["b"]),
                  ee["l2"]["w"], _row(ee["l2"]["b"]))

    blocks = params["blocks"]
    for k, bp in enumerate(blocks):
        hs, hd = _gather_two(h, src3, dst3)
        em = bp["edge_mlp"]
        e = _edge_update(e, hs, hd,
                         em["l1"]["w"], _row(em["l1"]["b"]),
                         em["l2"]["w"], _row(em["l2"]["b"]),
                         _row(bp["ln_e"]["g"]), _row(bp["ln_e"]["b"]))
        agg = _scatter_sum(e, dst3)
        nm = bp["node_mlp"]
        common = (h, agg[0], agg[1],
                  nm["l1"]["w"], _row(nm["l1"]["b"]),
                  nm["l2"]["w"], _row(nm["l2"]["b"]),
                  _row(bp["ln_h"]["g"]), _row(bp["ln_h"]["b"]))
        if k + 1 < len(blocks):
            h = _node_update(*common)
        else:
            nh = params["node_head"]
            h, nl = _node_last(
                *common,
                nh["l1"]["w"], _row(nh["l1"]["b"]),
                nh["l2"]["w"], _row(nh["l2"]["b"]))

    hs, hd = _gather_two(h, src3, dst3)
    hd_p = params["edge_head"]
    el = _edge_head(e, hs, hd,
                    hd_p["l1"]["w"], _row(hd_p["l1"]["b"]),
                    hd_p["l2"]["w"], _row(hd_p["l2"]["b"]))

    return (el[:, 0], nl[:, 0])
